# baseline (speedup 1.0000x reference)
import functools

import jax
import jax.numpy as jnp
from jax import lax
from jax.experimental import pallas as pl
from jax.experimental.pallas import tpu as pltpu
from jax.experimental.pallas import tpu_sc as plsc

N = 10000
D = 128
E = 320000

NC = 2
NS = 16
NW = NC * NS
BK = 128
NB = 80
EPT = NB * BK
EPAD = EPT * NW
NACC = 10240
RPS = NACC // NS

NQ = NACC // 4
DEAD = 64
NA2 = NQ + DEAD
RP2 = NA2 // NS
CAP = EPT + BK

_mesh = plsc.VectorSubcoreMesh(
    core_axis_name="c", subcore_axis_name="s", num_cores=NC, num_subcores=NS)
_sc_params = pltpu.CompilerParams(
    needs_layout_passes=False, use_tc_tiling_on_sc=False)


def _clipw(e16):
    a = jnp.abs(e16)
    a = jnp.where(a != a, jnp.float32(0.0), a)
    a = jnp.where(a == jnp.float32(jnp.inf), jnp.float32(0.0), a)
    return jnp.maximum(a, jnp.float32(1e-6))


@functools.partial(
    pl.kernel,
    out_type=jax.ShapeDtypeStruct((NC, NACC, 16), jnp.float32),
    mesh=_mesh,
    compiler_params=_sc_params,
    scratch_types=[
        pltpu.VMEM((NB, BK), jnp.int32),
        pltpu.VMEM((NB, BK), jnp.float32),
        pltpu.VMEM((BK, 16), jnp.float32),
        pltpu.VMEM((RPS, 16), jnp.float32),
        pltpu.VMEM_SHARED((NACC, 16), jnp.float32),
    ],
)
def _sc_deg(col3, ew3, deg_out, col_v, ew_v, msg_v, zero_v, acc):
    c = lax.axis_index("c")
    s = lax.axis_index("s")
    wid = c * NS + s
    zeros16 = jnp.zeros((16,), jnp.float32)

    def zrow(i, _):
        zero_v[i, :] = zeros16
        return 0
    lax.fori_loop(0, RPS, zrow, 0)
    pltpu.sync_copy(zero_v, acc.at[pl.ds(s * RPS, RPS)])
    pltpu.sync_copy(col3.at[wid], col_v)
    pltpu.sync_copy(ew3.at[wid], ew_v)
    plsc.subcore_barrier()

    iota16 = lax.iota(jnp.int32, 16)
    lanes0 = jnp.zeros((16,), jnp.int32)

    def zmsg(i, _):
        msg_v[i, :] = zeros16
        return 0
    lax.fori_loop(0, BK, zmsg, 0)

    def block(t, _):
        for g in range(8):
            e16 = ew_v[t, pl.ds(g * 16, 16)]
            plsc.store_scatter(msg_v, [iota16 + g * 16, lanes0], _clipw(e16))
        pltpu.sync_copy(msg_v, acc.at[col_v.at[t]], add=True)
        return 0
    lax.fori_loop(0, NB, block, 0)
    plsc.subcore_barrier()
    pltpu.sync_copy(acc.at[pl.ds(s * RPS, RPS)],
                    deg_out.at[c, pl.ds(s * RPS, RPS)])


def _make_part(lo):
  @functools.partial(
    pl.kernel,
    out_type=(
        jax.ShapeDtypeStruct((NC, NW, CAP), jnp.int32),
        jax.ShapeDtypeStruct((NC, NW, CAP), jnp.int32),
        jax.ShapeDtypeStruct((NC, NW, CAP), jnp.float32),
        jax.ShapeDtypeStruct((NC, NW, 16), jnp.int32),
    ),
    mesh=_mesh,
    compiler_params=_sc_params,
    scratch_types=[
        pltpu.VMEM((NB, BK), jnp.int32),
        pltpu.VMEM((NB, BK), jnp.int32),
        pltpu.VMEM((NB, BK), jnp.float32),
        pltpu.VMEM((CAP,), jnp.int32),
        pltpu.VMEM((CAP,), jnp.int32),
        pltpu.VMEM((CAP,), jnp.float32),
        pltpu.VMEM((CAP,), jnp.int32),
        pltpu.VMEM((CAP,), jnp.int32),
        pltpu.VMEM((CAP,), jnp.float32),
        pltpu.VMEM((16,), jnp.int32),
    ],
)
  def _part(row3, col3, ew3, rowP, colP, ewP, cnts,
            row_v, col_v, ew_v, b0r, b0c, b0w, b1r, b1c, b1w, cnt_v):
    c = lax.axis_index("c")
    s = lax.axis_index("s")
    wid = c * NS + s
    pltpu.sync_copy(row3.at[wid], row_v)
    pltpu.sync_copy(col3.at[wid], col_v)
    pltpu.sync_copy(ew3.at[wid], ew_v)

    iota16 = lax.iota(jnp.int32, 16)
    zero16f = jnp.zeros((16,), jnp.float32)
    zero16i = jnp.zeros((16,), jnp.int32)

    def blk(t, carry):
        n0, n1 = carry
        for g in range(8):
            c16 = col_v[t, pl.ds(g * 16, 16)]
            r16 = row_v[t, pl.ds(g * 16, 16)]
            e16 = ew_v[t, pl.ds(g * 16, 16)]
            cr = c16 - lo
            m0 = (cr >= 0) & (cr < NQ)
            m1 = (cr >= NQ) & (cr < 2 * NQ)
            plsc.store_compressed(b0c.at[pl.ds(n0, 16)], cr, mask=m0)
            plsc.store_compressed(b0r.at[pl.ds(n0, 16)], r16, mask=m0)
            plsc.store_compressed(b0w.at[pl.ds(n0, 16)], e16, mask=m0)
            plsc.store_compressed(b1c.at[pl.ds(n1, 16)], cr - NQ, mask=m1)
            plsc.store_compressed(b1r.at[pl.ds(n1, 16)], r16, mask=m1)
            plsc.store_compressed(b1w.at[pl.ds(n1, 16)], e16, mask=m1)
            n0 = n0 + plsc.all_reduce_population_count(m0)[0]
            n1 = n1 + plsc.all_reduce_population_count(m1)[0]
        return (n0, n1)
    n0, n1 = lax.fori_loop(0, NB, blk, (jnp.int32(0), jnp.int32(0)))

    for bc, br, bw, n in ((b0c, b0r, b0w, n0), (b1c, b1r, b1w, n1)):
        for j in range(8):
            bc[pl.ds(n + j * 16, 16)] = NQ + iota16 + (j % 4) * 16
            br[pl.ds(n + j * 16, 16)] = zero16i
            bw[pl.ds(n + j * 16, 16)] = zero16f
    n0p = ((n0 + BK - 1) // BK) * BK
    n1p = ((n1 + BK - 1) // BK) * BK

    cnt_v[:] = jnp.full((16,), n0p, jnp.int32)
    pltpu.sync_copy(cnt_v, cnts.at[0, wid])
    cnt_v[:] = jnp.full((16,), n1p, jnp.int32)
    pltpu.sync_copy(cnt_v, cnts.at[1, wid])
    pltpu.sync_copy(b0r, rowP.at[0, wid])
    pltpu.sync_copy(b0c, colP.at[0, wid])
    pltpu.sync_copy(b0w, ewP.at[0, wid])
    pltpu.sync_copy(b1r, rowP.at[1, wid])
    pltpu.sync_copy(b1c, colP.at[1, wid])
    pltpu.sync_copy(b1w, ewP.at[1, wid])
  return _part


_sc_part_a = _make_part(0)
_sc_part_b = _make_part(2 * NQ)


@functools.partial(
    pl.kernel,
    out_type=jax.ShapeDtypeStruct((NC, NA2, D), jnp.float32),
    mesh=_mesh,
    compiler_params=_sc_params,
    scratch_types=[
        pltpu.VMEM((CAP // BK, BK), jnp.int32),
        pltpu.VMEM((CAP // BK, BK), jnp.int32),
        pltpu.VMEM((CAP // BK, BK), jnp.float32),
        pltpu.VMEM((N,), jnp.float32),
        pltpu.VMEM((BK, D), jnp.float32),
        pltpu.VMEM((BK, D), jnp.float32),
        pltpu.VMEM((41, D), jnp.float32),
        pltpu.VMEM((16,), jnp.int32),
        pltpu.VMEM_SHARED((NA2, D), jnp.float32),
        pltpu.SemaphoreType.DMA,
        pltpu.SemaphoreType.DMA,
        pltpu.SemaphoreType.DMA,
    ],
)
def _sc_agg(hw, dinv, rowP, colP, ewP, cnts, out,
            lrow, lcol, lew, dinv_v, rows_0, rows_1,
            zero_v, cnt_v, acc,
            gsem_0, gsem_1, ssem):
    c = lax.axis_index("c")
    s = lax.axis_index("s")
    zeros16 = jnp.zeros((16,), jnp.float32)

    def zrow(i, _):
        for g in range(8):
            zero_v[i, pl.ds(g * 16, 16)] = zeros16
        return 0
    lax.fori_loop(0, 41, zrow, 0)

    def zacc(k, _):
        pltpu.sync_copy(zero_v, acc.at[pl.ds(s * RP2 + k * 41, 41)])
        return 0
    lax.fori_loop(0, RP2 // 41, zacc, 0)
    pltpu.sync_copy(dinv, dinv_v)
    plsc.subcore_barrier()

    bufs = (rows_0, rows_1)
    gsems = (gsem_0, gsem_1)

    def _scale(t, rows_v):
        def group(g, _):
            r16 = lrow[t, pl.ds(g * 16, 16)]
            e16 = lew[t, pl.ds(g * 16, 16)]
            w16 = _clipw(e16) * plsc.load_gather(dinv_v, [r16])
            for u in range(16):
                wspl = jnp.full((16,), w16[u])
                for q in range(D // 16):
                    rows_v[g * 16 + u, pl.ds(q * 16, 16)] = (
                        rows_v[g * 16 + u, pl.ds(q * 16, 16)] * wspl)
            return 0
        lax.fori_loop(0, 8, group, 0)

    def _gather(t, rows_v, gsem):
        pltpu.async_copy(hw.at[lrow.at[t]], rows_v, gsem)

    def _gwait(rows_v, gsem):
        pltpu.make_async_copy(hw.at[lrow.at[0]], rows_v, gsem).wait()

    def _scat(t, rows_v):
        pltpu.sync_copy(rows_v, acc.at[lcol.at[t]], add=True)

    for li in range(2):
        pid = s * 2 + li
        pltpu.sync_copy(cnts.at[c, pid], cnt_v)
        nblk = cnt_v[...][0] // BK
        pltpu.sync_copy(rowP.at[c, pid], lrow)
        pltpu.sync_copy(colP.at[c, pid], lcol)
        pltpu.sync_copy(ewP.at[c, pid], lew)
        for p in range(2):
            @pl.when(p < nblk)
            def _():
                _gather(p, bufs[p], gsems[p])

        def pair(i, _):
            t0 = i * 2
            for p in range(2):
                t = t0 + p

                @pl.when(t < nblk)
                def _():
                    _gwait(bufs[p], gsems[p])
                    _scale(t, bufs[p])
                    _scat(t, bufs[p])

                    @pl.when(t + 2 < nblk)
                    def _():
                        _gather(t + 2, bufs[p], gsems[p])
            return 0
        lax.fori_loop(0, (CAP // BK + 1) // 2, pair, 0)
    plsc.subcore_barrier()
    pltpu.sync_copy(acc.at[pl.ds(s * RP2, RP2)],
                    out.at[c, pl.ds(s * RP2, RP2)])


BN = 1000


def _tc_prep_body(x_ref, w1_ref, deg0_ref, deg1_ref,
                  hw_ref, dinv_ref, dself_ref):
    xs = jnp.nan_to_num(x_ref[...])
    hw_ref[...] = jnp.dot(xs, w1_ref[...], preferred_element_type=jnp.float32)
    d = deg0_ref[...][:, :1] + deg1_ref[...][:, :1] + 1.0
    dinv_ref[...] = lax.rsqrt(d)
    dself_ref[...] = 1.0 / d


_tc_prep = pl.pallas_call(
    _tc_prep_body,
    grid=(N // BN,),
    in_specs=[
        pl.BlockSpec((BN, D), lambda i: (i, 0)),
        pl.BlockSpec((D, D), lambda i: (0, 0)),
        pl.BlockSpec((BN, 16), lambda i: (i, 0)),
        pl.BlockSpec((BN, 16), lambda i: (i, 0)),
    ],
    out_specs=[
        pl.BlockSpec((BN, D), lambda i: (i, 0)),
        pl.BlockSpec((BN, 1), lambda i: (i, 0)),
        pl.BlockSpec((BN, 1), lambda i: (i, 0)),
    ],
    out_shape=[
        jax.ShapeDtypeStruct((N, D), jnp.float32),
        jax.ShapeDtypeStruct((N, 1), jnp.float32),
        jax.ShapeDtypeStruct((N, 1), jnp.float32),
    ],
)


def _post_math(agg, hw, dinv, dself, b, g, be, res):
    conv = dinv * agg + dself * hw + b
    conv = jnp.nan_to_num(conv)
    mu = jnp.mean(conv, axis=-1, keepdims=True)
    var = jnp.mean((conv - mu) ** 2, axis=-1, keepdims=True)
    hn = (conv - mu) * lax.rsqrt(var + 1e-5) * g + be
    hn = jnp.nan_to_num(hn)
    return jnp.maximum(hn, 0.0) + jnp.nan_to_num(res)


def _tc_post1_body(agg_ref, hw_ref, dinv_ref, dself_ref,
                   b_ref, g_ref, be_ref, res_ref, w2_ref, h_ref, hw2_ref):
    h = _post_math(agg_ref[...], hw_ref[...], dinv_ref[...], dself_ref[...],
                   b_ref[...], g_ref[...], be_ref[...], res_ref[...])
    h_ref[...] = h
    hw2_ref[...] = jnp.dot(h, w2_ref[...], preferred_element_type=jnp.float32)


def _tc_post2_body(agg_ref, hw_ref, dinv_ref, dself_ref,
                   b_ref, g_ref, be_ref, res_ref, h_ref):
    h_ref[...] = _post_math(
        agg_ref[...], hw_ref[...], dinv_ref[...], dself_ref[...],
        b_ref[...], g_ref[...], be_ref[...], res_ref[...])


_post_in_specs = [
    pl.BlockSpec((BN, D), lambda i: (i, 0)),
    pl.BlockSpec((BN, D), lambda i: (i, 0)),
    pl.BlockSpec((BN, 1), lambda i: (i, 0)),
    pl.BlockSpec((BN, 1), lambda i: (i, 0)),
    pl.BlockSpec((1, D), lambda i: (0, 0)),
    pl.BlockSpec((1, D), lambda i: (0, 0)),
    pl.BlockSpec((1, D), lambda i: (0, 0)),
    pl.BlockSpec((BN, D), lambda i: (i, 0)),
]

_tc_post1 = pl.pallas_call(
    _tc_post1_body,
    grid=(N // BN,),
    in_specs=_post_in_specs + [pl.BlockSpec((D, D), lambda i: (0, 0))],
    out_specs=[
        pl.BlockSpec((BN, D), lambda i: (i, 0)),
        pl.BlockSpec((BN, D), lambda i: (i, 0)),
    ],
    out_shape=[
        jax.ShapeDtypeStruct((N, D), jnp.float32),
        jax.ShapeDtypeStruct((N, D), jnp.float32),
    ],
)

_tc_post2 = pl.pallas_call(
    _tc_post2_body,
    grid=(N // BN,),
    in_specs=_post_in_specs,
    out_specs=pl.BlockSpec((BN, D), lambda i: (i, 0)),
    out_shape=jax.ShapeDtypeStruct((N, D), jnp.float32),
)


def kernel(x, edge_index, edge_weight, W1, b1, g1, be1, W2, b2, g2, be2):
    row = edge_index[0].astype(jnp.int32)
    col = edge_index[1].astype(jnp.int32)
    ew = edge_weight.reshape(-1).astype(jnp.float32)
    pad = EPAD - E
    row3 = jnp.concatenate(
        [row, jnp.zeros((pad,), jnp.int32)]).reshape(NW, NB, BK)
    pad_col = N + jnp.arange(pad, dtype=jnp.int32) % (NACC - N)
    col3 = jnp.concatenate([col, pad_col]).reshape(NW, NB, BK)
    ew3 = jnp.concatenate(
        [ew, jnp.zeros((pad,), jnp.float32)]).reshape(NW, NB, BK)

    degp = _sc_deg(col3, ew3)
    partA = _sc_part_a(row3, col3, ew3)
    partB = _sc_part_b(row3, col3, ew3)
    hw1, dinv, dself = _tc_prep(x, W1, degp[0, :N], degp[1, :N])
    dinv_flat = dinv.reshape(N)

    b1r, g1r, be1r = b1.reshape(1, D), g1.reshape(1, D), be1.reshape(1, D)
    b2r, g2r, be2r = b2.reshape(1, D), g2.reshape(1, D), be2.reshape(1, D)

    def _r(p):
        rowP, colP, ewP, cnts = p
        sh = (NC, NW, CAP // BK, BK)
        return rowP.reshape(sh), colP.reshape(sh), ewP.reshape(sh), cnts

    partA = _r(partA)
    partB = _r(partB)

    def agg_layer(hw):
        oa = _sc_agg(hw, dinv_flat, *partA)
        ob = _sc_agg(hw, dinv_flat, *partB)
        return jnp.concatenate(
            [oa[0, :NQ], oa[1, :NQ], ob[0, :NQ], ob[1, :NQ]], axis=0)[:N]

    a1 = agg_layer(hw1)
    h1, hw2 = _tc_post1(a1, hw1, dinv, dself, b1r, g1r, be1r, x, W2)
    a2 = agg_layer(hw2)
    h2 = _tc_post2(a2, hw2, dinv, dself, b2r, g2r, be2r, h1)
    return h2


# speedup vs baseline: 1.5978x; 1.5978x over previous
import functools

import jax
import jax.numpy as jnp
from jax import lax
from jax.experimental import pallas as pl
from jax.experimental.pallas import tpu as pltpu
from jax.experimental.pallas import tpu_sc as plsc

N = 10000
D = 128
HD = 64
E = 320000

NC = 2
NS = 16
NW = NC * NS
BK = 128
NB0 = 100
NB1 = 60
NB = NB0
EPAD = (NB0 + NB1) * NS * BK
NACC = 10240
RPS = NACC // NS

_mesh = plsc.VectorSubcoreMesh(
    core_axis_name="c", subcore_axis_name="s", num_cores=NC, num_subcores=NS)
_sc_params = pltpu.CompilerParams(needs_layout_passes=False, use_tc_tiling_on_sc=False)


def _clipw(e16):
    a = jnp.abs(e16)
    a = jnp.where(a != a, jnp.float32(0.0), a)
    a = jnp.where(a == jnp.float32(jnp.inf), jnp.float32(0.0), a)
    return jnp.maximum(a, jnp.float32(1e-6))


@functools.partial(
    pl.kernel,
    out_type=jax.ShapeDtypeStruct((NC, NACC, 16), jnp.float32),
    mesh=_mesh,
    compiler_params=_sc_params,
    scratch_types=[
        pltpu.VMEM((NB, BK), jnp.int32),
        pltpu.VMEM((NB, BK), jnp.float32),
        pltpu.VMEM((BK, 16), jnp.float32),
        pltpu.VMEM((RPS, 16), jnp.float32),
        pltpu.VMEM_SHARED((NACC, 16), jnp.float32),
    ],
)
def _sc_deg(col3, ew3, deg_out, col_v, ew_v, msg_v, zero_v, acc):
    c = lax.axis_index("c")
    s = lax.axis_index("s")
    wid = c * NS + s
    zeros16 = jnp.zeros((16,), jnp.float32)

    def zrow(i, _):
        zero_v[i, :] = zeros16
        return 0
    lax.fori_loop(0, RPS, zrow, 0)
    pltpu.sync_copy(zero_v, acc.at[pl.ds(s * RPS, RPS)])
    pltpu.sync_copy(col3.at[wid], col_v)
    pltpu.sync_copy(ew3.at[wid], ew_v)
    plsc.subcore_barrier()

    iota16 = lax.iota(jnp.int32, 16)
    lanes0 = jnp.zeros((16,), jnp.int32)

    def zmsg(i, _):
        msg_v[i, :] = zeros16
        return 0
    lax.fori_loop(0, BK, zmsg, 0)

    def block(t, _):
        for g in range(8):
            e16 = ew_v[t, pl.ds(g * 16, 16)]
            plsc.store_scatter(msg_v, [iota16 + g * 16, lanes0], _clipw(e16))
        pltpu.sync_copy(msg_v, acc.at[col_v.at[t]], add=True)
        return 0
    lax.fori_loop(0, NB, block, 0)
    plsc.subcore_barrier()
    pltpu.sync_copy(acc.at[pl.ds(s * RPS, RPS)],
                    deg_out.at[c, pl.ds(s * RPS, RPS)])


@functools.partial(
    pl.kernel,
    out_type=jax.ShapeDtypeStruct((NC, NACC, HD), jnp.float32),
    mesh=_mesh,
    compiler_params=_sc_params,
    scratch_types=[
        pltpu.VMEM((NB, BK), jnp.int32),
        pltpu.VMEM((NB, BK), jnp.int32),
        pltpu.VMEM((NB, BK), jnp.float32),
        pltpu.VMEM((N,), jnp.float32),
        pltpu.VMEM((BK,), jnp.float32),
        pltpu.VMEM((BK, HD), jnp.float32),
        pltpu.VMEM((BK, HD), jnp.float32),
        pltpu.VMEM((BK, HD), jnp.float32),
        pltpu.VMEM((BK, HD), jnp.float32),
        pltpu.VMEM((64, HD), jnp.float32),
        pltpu.VMEM_SHARED((NACC, HD), jnp.float32),
        pltpu.SemaphoreType.DMA,
        pltpu.SemaphoreType.DMA,
        pltpu.SemaphoreType.DMA,
        pltpu.SemaphoreType.DMA,
        pltpu.SemaphoreType.DMA,
        pltpu.SemaphoreType.DMA,
        pltpu.SemaphoreType.DMA,
        pltpu.SemaphoreType.DMA,
    ],
)
def _sc_agg(hw, dinv, row3, col3, ew3, out,
            row_v, col_v, ew_v, dinv_v, wblk_v, rows_0, rows_1, rows_2,
            rows_3, zero_v, acc,
            gsem_0, gsem_1, gsem_2, gsem_3, ssem_0, ssem_1, ssem_2, ssem_3):
    c = lax.axis_index("c")
    s = lax.axis_index("s")
    wid = c * NS + s
    nb = jnp.where(c == 0, jnp.int32(NB0), jnp.int32(NB1))
    zeros16 = jnp.zeros((16,), jnp.float32)
    NG = HD // 16

    def zrow(i, _):
        for g in range(NG):
            zero_v[i, pl.ds(g * 16, 16)] = zeros16
        return 0
    lax.fori_loop(0, 64, zrow, 0)

    def zacc(k, _):
        pltpu.sync_copy(zero_v, acc.at[pl.ds(s * RPS + k * 64, 64)])
        return 0
    lax.fori_loop(0, RPS // 64, zacc, 0)

    pltpu.sync_copy(row3.at[wid], row_v)
    pltpu.sync_copy(col3.at[wid], col_v)
    pltpu.sync_copy(ew3.at[wid], ew_v)
    pltpu.sync_copy(dinv, dinv_v)
    plsc.subcore_barrier()

    def _scale(t, rows_v):
        def group(g, _):
            r16 = row_v[t, pl.ds(g * 16, 16)]
            e16 = ew_v[t, pl.ds(g * 16, 16)]
            w16 = _clipw(e16) * plsc.load_gather(dinv_v, [r16])
            for u in range(16):
                wspl = jnp.full((16,), w16[u])
                e = u
                for q in range(HD // 16):
                    rows_v[g * 16 + e, pl.ds(q * 16, 16)] = (
                        rows_v[g * 16 + e, pl.ds(q * 16, 16)] * wspl)
            return 0
        lax.fori_loop(0, 8, group, 0)

    def _gather(t, rows_v, gsem):
        pltpu.async_copy(hw.at[row_v.at[t]], rows_v, gsem)

    def _gwait(rows_v, gsem):
        pltpu.make_async_copy(hw.at[row_v.at[0]], rows_v, gsem).wait()

    def _scat(t, rows_v, ssem):
        pltpu.async_copy(rows_v, acc.at[col_v.at[t]], ssem, add=True)

    def _swait(rows_v, ssem):
        pltpu.make_async_copy(rows_v, acc.at[col_v.at[0]], ssem).wait()

    bufs = (rows_0, rows_1, rows_2, rows_3)
    gsems = (gsem_0, gsem_1, gsem_2, gsem_3)
    ssems = (ssem_0, ssem_1, ssem_2, ssem_3)
    for p in range(4):
        _gather(p, bufs[p], gsems[p])

    def quad(i, _):
        t0 = i * 4
        for p in range(4):
            _gwait(bufs[p], gsems[p])
            _scale(t0 + p, bufs[p])
            _scat(t0 + p, bufs[p], ssems[p])

            @pl.when(t0 + p + 4 < nb)
            def _():
                _swait(bufs[p], ssems[p])
                _gather(t0 + p + 4, bufs[p], gsems[p])
        return 0
    lax.fori_loop(0, nb // 4, quad, 0)
    for p in range(4):
        _swait(bufs[p], ssems[p])
    plsc.subcore_barrier()
    pltpu.sync_copy(acc.at[pl.ds(s * RPS, RPS)],
                    out.at[c, pl.ds(s * RPS, RPS)])


BN = 1000


def _tc_prep_body(x_ref, w1_ref, deg0_ref, deg1_ref,
                  hwlo_ref, hwhi_ref, dinv_ref, dself_ref):
    xs = jnp.nan_to_num(x_ref[...])
    hw = jnp.dot(xs, w1_ref[...], preferred_element_type=jnp.float32)
    hwlo_ref[...] = hw[:, :HD]
    hwhi_ref[...] = hw[:, HD:]
    d = deg0_ref[...][:, :1] + deg1_ref[...][:, :1] + 1.0
    dinv_ref[...] = lax.rsqrt(d)
    dself_ref[...] = 1.0 / d


_tc_prep = pl.pallas_call(
    _tc_prep_body,
    grid=(N // BN,),
    in_specs=[
        pl.BlockSpec((BN, D), lambda i: (i, 0)),
        pl.BlockSpec((D, D), lambda i: (0, 0)),
        pl.BlockSpec((BN, 16), lambda i: (i, 0)),
        pl.BlockSpec((BN, 16), lambda i: (i, 0)),
    ],
    out_specs=[
        pl.BlockSpec((BN, HD), lambda i: (i, 0)),
        pl.BlockSpec((BN, HD), lambda i: (i, 0)),
        pl.BlockSpec((BN, 1), lambda i: (i, 0)),
        pl.BlockSpec((BN, 1), lambda i: (i, 0)),
    ],
    out_shape=[
        jax.ShapeDtypeStruct((N, HD), jnp.float32),
        jax.ShapeDtypeStruct((N, HD), jnp.float32),
        jax.ShapeDtypeStruct((N, 1), jnp.float32),
        jax.ShapeDtypeStruct((N, 1), jnp.float32),
    ],
)


def _post_math(alo0, alo1, ahi0, ahi1, hwlo, hwhi, dinv, dself, b, g, be, res):
    agg = jnp.concatenate([alo0 + alo1, ahi0 + ahi1], axis=1)
    hw = jnp.concatenate([hwlo, hwhi], axis=1)
    conv = dinv * agg + dself * hw + b
    conv = jnp.nan_to_num(conv)
    mu = jnp.mean(conv, axis=-1, keepdims=True)
    var = jnp.mean((conv - mu) ** 2, axis=-1, keepdims=True)
    hn = (conv - mu) * lax.rsqrt(var + 1e-5) * g + be
    hn = jnp.nan_to_num(hn)
    return jnp.maximum(hn, 0.0) + jnp.nan_to_num(res)


def _tc_post1_body(alo0_ref, alo1_ref, ahi0_ref, ahi1_ref, hwlo_ref, hwhi_ref,
                   dinv_ref, dself_ref, b_ref, g_ref, be_ref, res_ref, w2_ref,
                   h_ref, hw2lo_ref, hw2hi_ref):
    h = _post_math(alo0_ref[...], alo1_ref[...], ahi0_ref[...], ahi1_ref[...],
                   hwlo_ref[...], hwhi_ref[...], dinv_ref[...], dself_ref[...],
                   b_ref[...], g_ref[...], be_ref[...], res_ref[...])
    h_ref[...] = h
    hw2 = jnp.dot(h, w2_ref[...], preferred_element_type=jnp.float32)
    hw2lo_ref[...] = hw2[:, :HD]
    hw2hi_ref[...] = hw2[:, HD:]


def _tc_post2_body(alo0_ref, alo1_ref, ahi0_ref, ahi1_ref, hwlo_ref, hwhi_ref,
                   dinv_ref, dself_ref, b_ref, g_ref, be_ref, res_ref, h_ref):
    h_ref[...] = _post_math(
        alo0_ref[...], alo1_ref[...], ahi0_ref[...], ahi1_ref[...],
        hwlo_ref[...], hwhi_ref[...], dinv_ref[...], dself_ref[...],
        b_ref[...], g_ref[...], be_ref[...], res_ref[...])


_post_in_specs = [
    pl.BlockSpec((BN, HD), lambda i: (i, 0)),
    pl.BlockSpec((BN, HD), lambda i: (i, 0)),
    pl.BlockSpec((BN, HD), lambda i: (i, 0)),
    pl.BlockSpec((BN, HD), lambda i: (i, 0)),
    pl.BlockSpec((BN, HD), lambda i: (i, 0)),
    pl.BlockSpec((BN, HD), lambda i: (i, 0)),
    pl.BlockSpec((BN, 1), lambda i: (i, 0)),
    pl.BlockSpec((BN, 1), lambda i: (i, 0)),
    pl.BlockSpec((1, D), lambda i: (0, 0)),
    pl.BlockSpec((1, D), lambda i: (0, 0)),
    pl.BlockSpec((1, D), lambda i: (0, 0)),
    pl.BlockSpec((BN, D), lambda i: (i, 0)),
]

_tc_post1 = pl.pallas_call(
    _tc_post1_body,
    grid=(N // BN,),
    in_specs=_post_in_specs + [pl.BlockSpec((D, D), lambda i: (0, 0))],
    out_specs=[
        pl.BlockSpec((BN, D), lambda i: (i, 0)),
        pl.BlockSpec((BN, HD), lambda i: (i, 0)),
        pl.BlockSpec((BN, HD), lambda i: (i, 0)),
    ],
    out_shape=[
        jax.ShapeDtypeStruct((N, D), jnp.float32),
        jax.ShapeDtypeStruct((N, HD), jnp.float32),
        jax.ShapeDtypeStruct((N, HD), jnp.float32),
    ],
)

_tc_post2 = pl.pallas_call(
    _tc_post2_body,
    grid=(N // BN,),
    in_specs=_post_in_specs,
    out_specs=pl.BlockSpec((BN, D), lambda i: (i, 0)),
    out_shape=jax.ShapeDtypeStruct((N, D), jnp.float32),
)


def kernel(x, edge_index, edge_weight, W1, b1, g1, be1, W2, b2, g2, be2):
    row = edge_index[0].astype(jnp.int32)
    col = edge_index[1].astype(jnp.int32)
    ew = edge_weight.reshape(-1).astype(jnp.float32)
    pad = EPAD - E
    pad_col = N + jnp.arange(pad, dtype=jnp.int32) % (NACC - N)
    rowf = jnp.concatenate([row, jnp.zeros((pad,), jnp.int32)])
    colf = jnp.concatenate([col, pad_col])
    ewf = jnp.concatenate([ew, jnp.zeros((pad,), jnp.float32)])

    def _split(a, nullv):
        ea = a[:NS * NB0 * BK].reshape(NS, NB0, BK)
        eb = a[NS * NB0 * BK:].reshape(NS, NB1, BK)
        fill = jnp.full((NS, NB0 - NB1, BK), nullv, a.dtype)
        return jnp.concatenate([ea, jnp.concatenate([eb, fill], axis=1)],
                               axis=0)

    row3 = _split(rowf, 0)
    col3 = _split(colf, NACC - 1)
    ew3 = _split(ewf, 0)

    degp = _sc_deg(col3, ew3)
    hw1lo, hw1hi, dinv, dself = _tc_prep(x, W1, degp[0, :N], degp[1, :N])
    dinv_flat = dinv.reshape(N)

    b1r, g1r, be1r = b1.reshape(1, D), g1.reshape(1, D), be1.reshape(1, D)
    b2r, g2r, be2r = b2.reshape(1, D), g2.reshape(1, D), be2.reshape(1, D)

    agg1lo = _sc_agg(hw1lo, dinv_flat, row3, col3, ew3)
    agg1hi = _sc_agg(hw1hi, dinv_flat, row3, col3, ew3)
    h1, hw2lo, hw2hi = _tc_post1(
        agg1lo[0, :N], agg1lo[1, :N], agg1hi[0, :N], agg1hi[1, :N],
        hw1lo, hw1hi, dinv, dself, b1r, g1r, be1r, x, W2)
    agg2lo = _sc_agg(hw2lo, dinv_flat, row3, col3, ew3)
    agg2hi = _sc_agg(hw2hi, dinv_flat, row3, col3, ew3)
    h2 = _tc_post2(
        agg2lo[0, :N], agg2lo[1, :N], agg2hi[0, :N], agg2hi[1, :N],
        hw2lo, hw2hi, dinv, dself, b2r, g2r, be2r, h1)
    return h2


# speedup vs baseline: 1.6334x; 1.0223x over previous
import functools

import jax
import jax.numpy as jnp
from jax import lax
from jax.experimental import pallas as pl
from jax.experimental.pallas import tpu as pltpu
from jax.experimental.pallas import tpu_sc as plsc

N = 10000
D = 128
HD = 64
E = 320000

NC = 2
NS = 16
NW = NC * NS
BK = 128
NB0 = 100
NB1 = 60
NB = NB0
EPAD = (NB0 + NB1) * NS * BK
NACC = 10240
RPS = NACC // NS

_mesh = plsc.VectorSubcoreMesh(
    core_axis_name="c", subcore_axis_name="s", num_cores=NC, num_subcores=NS)
_sc_params = pltpu.CompilerParams(needs_layout_passes=False, use_tc_tiling_on_sc=False)


def _clipw(e16):
    a = jnp.abs(e16)
    a = jnp.where(a != a, jnp.float32(0.0), a)
    a = jnp.where(a == jnp.float32(jnp.inf), jnp.float32(0.0), a)
    return jnp.maximum(a, jnp.float32(1e-6))


@functools.partial(
    pl.kernel,
    out_type=jax.ShapeDtypeStruct((NC, NACC, 16), jnp.float32),
    mesh=_mesh,
    compiler_params=_sc_params,
    scratch_types=[
        pltpu.VMEM((NB, BK), jnp.int32),
        pltpu.VMEM((NB, BK), jnp.float32),
        pltpu.VMEM((BK, 16), jnp.float32),
        pltpu.VMEM((RPS, 16), jnp.float32),
        pltpu.VMEM_SHARED((NACC, 16), jnp.float32),
    ],
)
def _sc_deg(col3, ew3, deg_out, col_v, ew_v, msg_v, zero_v, acc):
    c = lax.axis_index("c")
    s = lax.axis_index("s")
    wid = c * NS + s
    zeros16 = jnp.zeros((16,), jnp.float32)

    def zrow(i, _):
        zero_v[i, :] = zeros16
        return 0
    lax.fori_loop(0, RPS, zrow, 0)
    pltpu.sync_copy(zero_v, acc.at[pl.ds(s * RPS, RPS)])
    pltpu.sync_copy(col3.at[wid], col_v)
    pltpu.sync_copy(ew3.at[wid], ew_v)
    plsc.subcore_barrier()

    iota16 = lax.iota(jnp.int32, 16)
    lanes0 = jnp.zeros((16,), jnp.int32)

    def zmsg(i, _):
        msg_v[i, :] = zeros16
        return 0
    lax.fori_loop(0, BK, zmsg, 0)

    def block(t, _):
        for g in range(8):
            e16 = ew_v[t, pl.ds(g * 16, 16)]
            plsc.store_scatter(msg_v, [iota16 + g * 16, lanes0], _clipw(e16))
        pltpu.sync_copy(msg_v, acc.at[col_v.at[t]], add=True)
        return 0
    lax.fori_loop(0, NB, block, 0)
    plsc.subcore_barrier()
    pltpu.sync_copy(acc.at[pl.ds(s * RPS, RPS)],
                    deg_out.at[c, pl.ds(s * RPS, RPS)])


@functools.partial(
    pl.kernel,
    out_type=jax.ShapeDtypeStruct((NC, NACC, HD), jnp.float32),
    mesh=_mesh,
    compiler_params=_sc_params,
    scratch_types=[
        pltpu.VMEM((NB, BK), jnp.int32),
        pltpu.VMEM((NB, BK), jnp.int32),
        pltpu.VMEM((NB, BK), jnp.float32),
        pltpu.VMEM((N,), jnp.float32),
        pltpu.VMEM((BK,), jnp.float32),
        pltpu.VMEM((BK, HD), jnp.float32),
        pltpu.VMEM((BK, HD), jnp.float32),
        pltpu.VMEM((BK, HD), jnp.float32),
        pltpu.VMEM((BK, HD), jnp.float32),
        pltpu.VMEM((64, HD), jnp.float32),
        pltpu.VMEM_SHARED((NACC, HD), jnp.float32),
        pltpu.SemaphoreType.DMA,
        pltpu.SemaphoreType.DMA,
        pltpu.SemaphoreType.DMA,
        pltpu.SemaphoreType.DMA,
        pltpu.SemaphoreType.DMA,
        pltpu.SemaphoreType.DMA,
        pltpu.SemaphoreType.DMA,
        pltpu.SemaphoreType.DMA,
    ],
)
def _sc_agg(hw, dinv, row3, col3, ew3, out,
            row_v, col_v, ew_v, dinv_v, wblk_v, rows_0, rows_1, rows_2,
            rows_3, zero_v, acc,
            gsem_0, gsem_1, gsem_2, gsem_3, ssem_0, ssem_1, ssem_2, ssem_3):
    c = lax.axis_index("c")
    s = lax.axis_index("s")
    wid = c * NS + s
    nb = jnp.where(c == 0, jnp.int32(NB1), jnp.int32(NB0))
    zeros16 = jnp.zeros((16,), jnp.float32)
    NG = HD // 16

    def zrow(i, _):
        for g in range(NG):
            zero_v[i, pl.ds(g * 16, 16)] = zeros16
        return 0
    lax.fori_loop(0, 64, zrow, 0)

    def zacc(k, _):
        pltpu.sync_copy(zero_v, acc.at[pl.ds(s * RPS + k * 64, 64)])
        return 0
    lax.fori_loop(0, RPS // 64, zacc, 0)

    pltpu.sync_copy(row3.at[wid], row_v)
    pltpu.sync_copy(col3.at[wid], col_v)
    pltpu.sync_copy(ew3.at[wid], ew_v)
    pltpu.sync_copy(dinv, dinv_v)
    plsc.subcore_barrier()

    def _scale(t, rows_v):
        def group(g, _):
            r16 = row_v[t, pl.ds(g * 16, 16)]
            e16 = ew_v[t, pl.ds(g * 16, 16)]
            w16 = _clipw(e16) * plsc.load_gather(dinv_v, [r16])
            for u in range(16):
                wspl = jnp.full((16,), w16[u])
                e = u
                for q in range(HD // 16):
                    rows_v[g * 16 + e, pl.ds(q * 16, 16)] = (
                        rows_v[g * 16 + e, pl.ds(q * 16, 16)] * wspl)
            return 0
        lax.fori_loop(0, 8, group, 0)

    def _gather(t, rows_v, gsem):
        pltpu.async_copy(hw.at[row_v.at[t]], rows_v, gsem)

    def _gwait(rows_v, gsem):
        pltpu.make_async_copy(hw.at[row_v.at[0]], rows_v, gsem).wait()

    def _scat(t, rows_v, ssem):
        pltpu.async_copy(rows_v, acc.at[col_v.at[t]], ssem, add=True)

    def _swait(rows_v, ssem):
        pltpu.make_async_copy(rows_v, acc.at[col_v.at[0]], ssem).wait()

    bufs = (rows_0, rows_1, rows_2, rows_3)
    gsems = (gsem_0, gsem_1, gsem_2, gsem_3)
    ssems = (ssem_0, ssem_1, ssem_2, ssem_3)
    for p in range(4):
        _gather(p, bufs[p], gsems[p])

    def quad(i, _):
        t0 = i * 4
        for p in range(4):
            _gwait(bufs[p], gsems[p])
            _scale(t0 + p, bufs[p])
            _scat(t0 + p, bufs[p], ssems[p])

            @pl.when(t0 + p + 4 < nb)
            def _():
                _swait(bufs[p], ssems[p])
                _gather(t0 + p + 4, bufs[p], gsems[p])
        return 0
    lax.fori_loop(0, nb // 4, quad, 0)
    for p in range(4):
        _swait(bufs[p], ssems[p])
    plsc.subcore_barrier()
    pltpu.sync_copy(acc.at[pl.ds(s * RPS, RPS)],
                    out.at[c, pl.ds(s * RPS, RPS)])


BN = 1000


def _tc_prep_body(x_ref, w1_ref, deg0_ref, deg1_ref,
                  hwlo_ref, hwhi_ref, dinv_ref, dself_ref):
    xs = jnp.nan_to_num(x_ref[...])
    hw = jnp.dot(xs, w1_ref[...], preferred_element_type=jnp.float32)
    hwlo_ref[...] = hw[:, :HD]
    hwhi_ref[...] = hw[:, HD:]
    d = deg0_ref[...][:, :1] + deg1_ref[...][:, :1] + 1.0
    dinv_ref[...] = lax.rsqrt(d)
    dself_ref[...] = 1.0 / d


_tc_prep = pl.pallas_call(
    _tc_prep_body,
    grid=(N // BN,),
    in_specs=[
        pl.BlockSpec((BN, D), lambda i: (i, 0)),
        pl.BlockSpec((D, D), lambda i: (0, 0)),
        pl.BlockSpec((BN, 16), lambda i: (i, 0)),
        pl.BlockSpec((BN, 16), lambda i: (i, 0)),
    ],
    out_specs=[
        pl.BlockSpec((BN, HD), lambda i: (i, 0)),
        pl.BlockSpec((BN, HD), lambda i: (i, 0)),
        pl.BlockSpec((BN, 1), lambda i: (i, 0)),
        pl.BlockSpec((BN, 1), lambda i: (i, 0)),
    ],
    out_shape=[
        jax.ShapeDtypeStruct((N, HD), jnp.float32),
        jax.ShapeDtypeStruct((N, HD), jnp.float32),
        jax.ShapeDtypeStruct((N, 1), jnp.float32),
        jax.ShapeDtypeStruct((N, 1), jnp.float32),
    ],
)


def _post_math(alo0, alo1, ahi0, ahi1, hwlo, hwhi, dinv, dself, b, g, be, res):
    agg = jnp.concatenate([alo0 + alo1, ahi0 + ahi1], axis=1)
    hw = jnp.concatenate([hwlo, hwhi], axis=1)
    conv = dinv * agg + dself * hw + b
    conv = jnp.nan_to_num(conv)
    mu = jnp.mean(conv, axis=-1, keepdims=True)
    var = jnp.mean((conv - mu) ** 2, axis=-1, keepdims=True)
    hn = (conv - mu) * lax.rsqrt(var + 1e-5) * g + be
    hn = jnp.nan_to_num(hn)
    return jnp.maximum(hn, 0.0) + jnp.nan_to_num(res)


def _tc_post1_body(alo0_ref, alo1_ref, ahi0_ref, ahi1_ref, hwlo_ref, hwhi_ref,
                   dinv_ref, dself_ref, b_ref, g_ref, be_ref, res_ref, w2_ref,
                   h_ref, hw2lo_ref, hw2hi_ref):
    h = _post_math(alo0_ref[...], alo1_ref[...], ahi0_ref[...], ahi1_ref[...],
                   hwlo_ref[...], hwhi_ref[...], dinv_ref[...], dself_ref[...],
                   b_ref[...], g_ref[...], be_ref[...], res_ref[...])
    h_ref[...] = h
    hw2 = jnp.dot(h, w2_ref[...], preferred_element_type=jnp.float32)
    hw2lo_ref[...] = hw2[:, :HD]
    hw2hi_ref[...] = hw2[:, HD:]


def _tc_post2_body(alo0_ref, alo1_ref, ahi0_ref, ahi1_ref, hwlo_ref, hwhi_ref,
                   dinv_ref, dself_ref, b_ref, g_ref, be_ref, res_ref, h_ref):
    h_ref[...] = _post_math(
        alo0_ref[...], alo1_ref[...], ahi0_ref[...], ahi1_ref[...],
        hwlo_ref[...], hwhi_ref[...], dinv_ref[...], dself_ref[...],
        b_ref[...], g_ref[...], be_ref[...], res_ref[...])


_post_in_specs = [
    pl.BlockSpec((BN, HD), lambda i: (i, 0)),
    pl.BlockSpec((BN, HD), lambda i: (i, 0)),
    pl.BlockSpec((BN, HD), lambda i: (i, 0)),
    pl.BlockSpec((BN, HD), lambda i: (i, 0)),
    pl.BlockSpec((BN, HD), lambda i: (i, 0)),
    pl.BlockSpec((BN, HD), lambda i: (i, 0)),
    pl.BlockSpec((BN, 1), lambda i: (i, 0)),
    pl.BlockSpec((BN, 1), lambda i: (i, 0)),
    pl.BlockSpec((1, D), lambda i: (0, 0)),
    pl.BlockSpec((1, D), lambda i: (0, 0)),
    pl.BlockSpec((1, D), lambda i: (0, 0)),
    pl.BlockSpec((BN, D), lambda i: (i, 0)),
]

_tc_post1 = pl.pallas_call(
    _tc_post1_body,
    grid=(N // BN,),
    in_specs=_post_in_specs + [pl.BlockSpec((D, D), lambda i: (0, 0))],
    out_specs=[
        pl.BlockSpec((BN, D), lambda i: (i, 0)),
        pl.BlockSpec((BN, HD), lambda i: (i, 0)),
        pl.BlockSpec((BN, HD), lambda i: (i, 0)),
    ],
    out_shape=[
        jax.ShapeDtypeStruct((N, D), jnp.float32),
        jax.ShapeDtypeStruct((N, HD), jnp.float32),
        jax.ShapeDtypeStruct((N, HD), jnp.float32),
    ],
)

_tc_post2 = pl.pallas_call(
    _tc_post2_body,
    grid=(N // BN,),
    in_specs=_post_in_specs,
    out_specs=pl.BlockSpec((BN, D), lambda i: (i, 0)),
    out_shape=jax.ShapeDtypeStruct((N, D), jnp.float32),
)


def kernel(x, edge_index, edge_weight, W1, b1, g1, be1, W2, b2, g2, be2):
    row = edge_index[0].astype(jnp.int32)
    col = edge_index[1].astype(jnp.int32)
    ew = edge_weight.reshape(-1).astype(jnp.float32)
    pad = EPAD - E
    pad_col = N + jnp.arange(pad, dtype=jnp.int32) % (NACC - N)
    rowf = jnp.concatenate([row, jnp.zeros((pad,), jnp.int32)])
    colf = jnp.concatenate([col, pad_col])
    ewf = jnp.concatenate([ew, jnp.zeros((pad,), jnp.float32)])

    def _split(a, nullv):
        ea = a[:NS * NB0 * BK].reshape(NS, NB0, BK)
        eb = a[NS * NB0 * BK:].reshape(NS, NB1, BK)
        fill = jnp.full((NS, NB0 - NB1, BK), nullv, a.dtype)
        return jnp.concatenate([jnp.concatenate([eb, fill], axis=1), ea],
                               axis=0)

    row3 = _split(rowf, 0)
    col3 = _split(colf, NACC - 1)
    ew3 = _split(ewf, 0)

    degp = _sc_deg(col3, ew3)
    hw1lo, hw1hi, dinv, dself = _tc_prep(x, W1, degp[0, :N], degp[1, :N])
    dinv_flat = dinv.reshape(N)

    b1r, g1r, be1r = b1.reshape(1, D), g1.reshape(1, D), be1.reshape(1, D)
    b2r, g2r, be2r = b2.reshape(1, D), g2.reshape(1, D), be2.reshape(1, D)

    agg1lo = _sc_agg(hw1lo, dinv_flat, row3, col3, ew3)
    agg1hi = _sc_agg(hw1hi, dinv_flat, row3, col3, ew3)
    h1, hw2lo, hw2hi = _tc_post1(
        agg1lo[0, :N], agg1lo[1, :N], agg1hi[0, :N], agg1hi[1, :N],
        hw1lo, hw1hi, dinv, dself, b1r, g1r, be1r, x, W2)
    agg2lo = _sc_agg(hw2lo, dinv_flat, row3, col3, ew3)
    agg2hi = _sc_agg(hw2hi, dinv_flat, row3, col3, ew3)
    h2 = _tc_post2(
        agg2lo[0, :N], agg2lo[1, :N], agg2hi[0, :N], agg2hi[1, :N],
        hw2lo, hw2hi, dinv, dself, b2r, g2r, be2r, h1)
    return h2


# speedup vs baseline: 1.7117x; 1.0479x over previous
import functools

import jax
import jax.numpy as jnp
from jax import lax
from jax.experimental import pallas as pl
from jax.experimental.pallas import tpu as pltpu
from jax.experimental.pallas import tpu_sc as plsc

N = 10000
D = 128
HD = 64
E = 320000

NC = 2
NS = 16
NW = NC * NS
BK = 128
NB = 80
EPT = NB * BK
EPAD = EPT * NW
NACC = 10240
RPS = NACC // NS

_mesh = plsc.VectorSubcoreMesh(
    core_axis_name="c", subcore_axis_name="s", num_cores=NC, num_subcores=NS)
_sc_params = pltpu.CompilerParams(needs_layout_passes=False, use_tc_tiling_on_sc=False)


def _clipw(e16):
    a = jnp.abs(e16)
    a = jnp.where(a != a, jnp.float32(0.0), a)
    a = jnp.where(a == jnp.float32(jnp.inf), jnp.float32(0.0), a)
    return jnp.maximum(a, jnp.float32(1e-6))


@functools.partial(
    pl.kernel,
    out_type=jax.ShapeDtypeStruct((NC, NACC, 16), jnp.float32),
    mesh=_mesh,
    compiler_params=_sc_params,
    scratch_types=[
        pltpu.VMEM((NB, BK), jnp.int32),
        pltpu.VMEM((NB, BK), jnp.float32),
        pltpu.VMEM((BK, 16), jnp.float32),
        pltpu.VMEM((RPS, 16), jnp.float32),
        pltpu.VMEM_SHARED((NACC, 16), jnp.float32),
    ],
)
def _sc_deg(col3, ew3, deg_out, col_v, ew_v, msg_v, zero_v, acc):
    c = lax.axis_index("c")
    s = lax.axis_index("s")
    wid = c * NS + s
    zeros16 = jnp.zeros((16,), jnp.float32)

    def zrow(i, _):
        zero_v[i, :] = zeros16
        return 0
    lax.fori_loop(0, RPS, zrow, 0)
    pltpu.sync_copy(zero_v, acc.at[pl.ds(s * RPS, RPS)])
    pltpu.sync_copy(col3.at[wid], col_v)
    pltpu.sync_copy(ew3.at[wid], ew_v)
    plsc.subcore_barrier()

    iota16 = lax.iota(jnp.int32, 16)
    lanes0 = jnp.zeros((16,), jnp.int32)

    def zmsg(i, _):
        msg_v[i, :] = zeros16
        return 0
    lax.fori_loop(0, BK, zmsg, 0)

    def block(t, _):
        for g in range(8):
            e16 = ew_v[t, pl.ds(g * 16, 16)]
            plsc.store_scatter(msg_v, [iota16 + g * 16, lanes0], _clipw(e16))
        pltpu.sync_copy(msg_v, acc.at[col_v.at[t]], add=True)
        return 0
    lax.fori_loop(0, NB, block, 0)
    plsc.subcore_barrier()
    pltpu.sync_copy(acc.at[pl.ds(s * RPS, RPS)],
                    deg_out.at[c, pl.ds(s * RPS, RPS)])


@functools.partial(
    pl.kernel,
    out_type=jax.ShapeDtypeStruct((NC, NACC, HD), jnp.float32),
    mesh=_mesh,
    compiler_params=_sc_params,
    scratch_types=[
        pltpu.VMEM((NB, BK), jnp.int32),
        pltpu.VMEM((NB, BK), jnp.int32),
        pltpu.VMEM((NB, BK), jnp.float32),
        pltpu.VMEM((N,), jnp.float32),
        pltpu.VMEM((BK,), jnp.float32),
        pltpu.VMEM((BK, HD), jnp.float32),
        pltpu.VMEM((BK, HD), jnp.float32),
        pltpu.VMEM((BK, HD), jnp.float32),
        pltpu.VMEM((BK, HD), jnp.float32),
        pltpu.VMEM((64, HD), jnp.float32),
        pltpu.VMEM_SHARED((NACC, HD), jnp.float32),
        pltpu.SemaphoreType.DMA,
        pltpu.SemaphoreType.DMA,
        pltpu.SemaphoreType.DMA,
        pltpu.SemaphoreType.DMA,
        pltpu.SemaphoreType.DMA,
        pltpu.SemaphoreType.DMA,
        pltpu.SemaphoreType.DMA,
        pltpu.SemaphoreType.DMA,
    ],
)
def _sc_agg(hw, dinv, row3, col3, ew3, out,
            row_v, col_v, ew_v, dinv_v, wblk_v, rows_0, rows_1, rows_2,
            rows_3, zero_v, acc,
            gsem_0, gsem_1, gsem_2, gsem_3, ssem_0, ssem_1, ssem_2, ssem_3):
    c = lax.axis_index("c")
    s = lax.axis_index("s")
    wid = c * NS + s
    zeros16 = jnp.zeros((16,), jnp.float32)
    NG = HD // 16

    def zrow(i, _):
        for g in range(NG):
            zero_v[i, pl.ds(g * 16, 16)] = zeros16
        return 0
    lax.fori_loop(0, 64, zrow, 0)

    def zacc(k, _):
        pltpu.sync_copy(zero_v, acc.at[pl.ds(s * RPS + k * 64, 64)])
        return 0
    lax.fori_loop(0, RPS // 64, zacc, 0)

    pltpu.sync_copy(row3.at[wid], row_v)
    pltpu.sync_copy(col3.at[wid], col_v)
    pltpu.sync_copy(ew3.at[wid], ew_v)
    pltpu.sync_copy(dinv, dinv_v)
    plsc.subcore_barrier()

    def _scale(t, rows_v):
        def group(g, _):
            r16 = row_v[t, pl.ds(g * 16, 16)]
            e16 = ew_v[t, pl.ds(g * 16, 16)]
            w16 = _clipw(e16) * plsc.load_gather(dinv_v, [r16])
            for u in range(16):
                wspl = jnp.full((16,), w16[u])
                e = u
                for q in range(HD // 16):
                    rows_v[g * 16 + e, pl.ds(q * 16, 16)] = (
                        rows_v[g * 16 + e, pl.ds(q * 16, 16)] * wspl)
            return 0
        lax.fori_loop(0, 8, group, 0)

    def _gather(t, rows_v, gsem):
        pltpu.async_copy(hw.at[row_v.at[t]], rows_v, gsem)

    def _gwait(rows_v, gsem):
        pltpu.make_async_copy(hw.at[row_v.at[0]], rows_v, gsem).wait()

    def _scat(t, rows_v, ssem):
        pltpu.async_copy(rows_v, acc.at[col_v.at[t]], ssem, add=True)

    def _swait(rows_v, ssem):
        pltpu.make_async_copy(rows_v, acc.at[col_v.at[0]], ssem).wait()

    bufs = (rows_0, rows_1, rows_2, rows_3)
    gsems = (gsem_0, gsem_1, gsem_2, gsem_3)
    ssems = (ssem_0, ssem_1, ssem_2, ssem_3)
    for p in range(4):
        _gather(p, bufs[p], gsems[p])

    def quad(i, _):
        t0 = i * 4
        for p in range(4):
            _gwait(bufs[p], gsems[p])
            _scale(t0 + p, bufs[p])
            _scat(t0 + p, bufs[p], ssems[p])

            @pl.when(i < NB // 4 - 1)
            def _():
                _swait(bufs[p], ssems[p])
                _gather(t0 + p + 4, bufs[p], gsems[p])
        return 0
    lax.fori_loop(0, NB // 4, quad, 0)
    for p in range(4):
        _swait(bufs[p], ssems[p])
    plsc.subcore_barrier()
    pltpu.sync_copy(acc.at[pl.ds(s * RPS, RPS)],
                    out.at[c, pl.ds(s * RPS, RPS)])


BN = 1000


def _tc_prep_body(x_ref, w1_ref, deg0_ref, deg1_ref,
                  hwlo_ref, hwhi_ref, dinv_ref, dself_ref):
    xs = jnp.nan_to_num(x_ref[...])
    hw = jnp.dot(xs, w1_ref[...], preferred_element_type=jnp.float32)
    hwlo_ref[...] = hw[:, :HD]
    hwhi_ref[...] = hw[:, HD:]
    d = deg0_ref[...][:, :1] + deg1_ref[...][:, :1] + 1.0
    dinv_ref[...] = lax.rsqrt(d)
    dself_ref[...] = 1.0 / d


_tc_prep = pl.pallas_call(
    _tc_prep_body,
    grid=(N // BN,),
    in_specs=[
        pl.BlockSpec((BN, D), lambda i: (i, 0)),
        pl.BlockSpec((D, D), lambda i: (0, 0)),
        pl.BlockSpec((BN, 16), lambda i: (i, 0)),
        pl.BlockSpec((BN, 16), lambda i: (i, 0)),
    ],
    out_specs=[
        pl.BlockSpec((BN, HD), lambda i: (i, 0)),
        pl.BlockSpec((BN, HD), lambda i: (i, 0)),
        pl.BlockSpec((BN, 1), lambda i: (i, 0)),
        pl.BlockSpec((BN, 1), lambda i: (i, 0)),
    ],
    out_shape=[
        jax.ShapeDtypeStruct((N, HD), jnp.float32),
        jax.ShapeDtypeStruct((N, HD), jnp.float32),
        jax.ShapeDtypeStruct((N, 1), jnp.float32),
        jax.ShapeDtypeStruct((N, 1), jnp.float32),
    ],
)


def _post_math(alo0, alo1, ahi0, ahi1, hwlo, hwhi, dinv, dself, b, g, be, res):
    agg = jnp.concatenate([alo0 + alo1, ahi0 + ahi1], axis=1)
    hw = jnp.concatenate([hwlo, hwhi], axis=1)
    conv = dinv * agg + dself * hw + b
    conv = jnp.nan_to_num(conv)
    mu = jnp.mean(conv, axis=-1, keepdims=True)
    var = jnp.mean((conv - mu) ** 2, axis=-1, keepdims=True)
    hn = (conv - mu) * lax.rsqrt(var + 1e-5) * g + be
    hn = jnp.nan_to_num(hn)
    return jnp.maximum(hn, 0.0) + jnp.nan_to_num(res)


def _tc_post1_body(alo0_ref, alo1_ref, ahi0_ref, ahi1_ref, hwlo_ref, hwhi_ref,
                   dinv_ref, dself_ref, b_ref, g_ref, be_ref, res_ref, w2_ref,
                   h_ref, hw2lo_ref, hw2hi_ref):
    h = _post_math(alo0_ref[...], alo1_ref[...], ahi0_ref[...], ahi1_ref[...],
                   hwlo_ref[...], hwhi_ref[...], dinv_ref[...], dself_ref[...],
                   b_ref[...], g_ref[...], be_ref[...], res_ref[...])
    h_ref[...] = h
    hw2 = jnp.dot(h, w2_ref[...], preferred_element_type=jnp.float32)
    hw2lo_ref[...] = hw2[:, :HD]
    hw2hi_ref[...] = hw2[:, HD:]


def _tc_post2_body(alo0_ref, alo1_ref, ahi0_ref, ahi1_ref, hwlo_ref, hwhi_ref,
                   dinv_ref, dself_ref, b_ref, g_ref, be_ref, res_ref, h_ref):
    h_ref[...] = _post_math(
        alo0_ref[...], alo1_ref[...], ahi0_ref[...], ahi1_ref[...],
        hwlo_ref[...], hwhi_ref[...], dinv_ref[...], dself_ref[...],
        b_ref[...], g_ref[...], be_ref[...], res_ref[...])


_post_in_specs = [
    pl.BlockSpec((BN, HD), lambda i: (i, 0)),
    pl.BlockSpec((BN, HD), lambda i: (i, 0)),
    pl.BlockSpec((BN, HD), lambda i: (i, 0)),
    pl.BlockSpec((BN, HD), lambda i: (i, 0)),
    pl.BlockSpec((BN, HD), lambda i: (i, 0)),
    pl.BlockSpec((BN, HD), lambda i: (i, 0)),
    pl.BlockSpec((BN, 1), lambda i: (i, 0)),
    pl.BlockSpec((BN, 1), lambda i: (i, 0)),
    pl.BlockSpec((1, D), lambda i: (0, 0)),
    pl.BlockSpec((1, D), lambda i: (0, 0)),
    pl.BlockSpec((1, D), lambda i: (0, 0)),
    pl.BlockSpec((BN, D), lambda i: (i, 0)),
]

_tc_post1 = pl.pallas_call(
    _tc_post1_body,
    grid=(N // BN,),
    in_specs=_post_in_specs + [pl.BlockSpec((D, D), lambda i: (0, 0))],
    out_specs=[
        pl.BlockSpec((BN, D), lambda i: (i, 0)),
        pl.BlockSpec((BN, HD), lambda i: (i, 0)),
        pl.BlockSpec((BN, HD), lambda i: (i, 0)),
    ],
    out_shape=[
        jax.ShapeDtypeStruct((N, D), jnp.float32),
        jax.ShapeDtypeStruct((N, HD), jnp.float32),
        jax.ShapeDtypeStruct((N, HD), jnp.float32),
    ],
)

_tc_post2 = pl.pallas_call(
    _tc_post2_body,
    grid=(N // BN,),
    in_specs=_post_in_specs,
    out_specs=pl.BlockSpec((BN, D), lambda i: (i, 0)),
    out_shape=jax.ShapeDtypeStruct((N, D), jnp.float32),
)


def kernel(x, edge_index, edge_weight, W1, b1, g1, be1, W2, b2, g2, be2):
    row = edge_index[0].astype(jnp.int32)
    col = edge_index[1].astype(jnp.int32)
    ew = edge_weight.reshape(-1).astype(jnp.float32)
    pad = EPAD - E
    row3 = jnp.concatenate(
        [row, jnp.zeros((pad,), jnp.int32)]).reshape(NW, NB, BK)
    pad_col = N + jnp.arange(pad, dtype=jnp.int32) % (NACC - N)
    col3 = jnp.concatenate([col, pad_col]).reshape(NW, NB, BK)
    ew3 = jnp.concatenate(
        [ew, jnp.zeros((pad,), jnp.float32)]).reshape(NW, NB, BK)

    degp = _sc_deg(col3, ew3)
    hw1lo, hw1hi, dinv, dself = _tc_prep(x, W1, degp[0, :N], degp[1, :N])
    dinv_flat = dinv.reshape(N)

    b1r, g1r, be1r = b1.reshape(1, D), g1.reshape(1, D), be1.reshape(1, D)
    b2r, g2r, be2r = b2.reshape(1, D), g2.reshape(1, D), be2.reshape(1, D)

    agg1lo = _sc_agg(hw1lo, dinv_flat, row3, col3, ew3)
    agg1hi = _sc_agg(hw1hi, dinv_flat, row3, col3, ew3)
    h1, hw2lo, hw2hi = _tc_post1(
        agg1lo[0, :N], agg1lo[1, :N], agg1hi[0, :N], agg1hi[1, :N],
        hw1lo, hw1hi, dinv, dself, b1r, g1r, be1r, x, W2)
    agg2lo = _sc_agg(hw2lo, dinv_flat, row3, col3, ew3)
    agg2hi = _sc_agg(hw2hi, dinv_flat, row3, col3, ew3)
    h2 = _tc_post2(
        agg2lo[0, :N], agg2lo[1, :N], agg2hi[0, :N], agg2hi[1, :N],
        hw2lo, hw2hi, dinv, dself, b2r, g2r, be2r, h1)
    return h2
